# baseline (speedup 1.0000x reference)
import functools

import jax
import jax.numpy as jnp
from jax import lax
from jax.experimental import pallas as pl
from jax.experimental.pallas import tpu as pltpu
from jax.experimental.pallas import tpu_sc as plsc

B = 16384
L = 50
D = 128
NC = 2
NS = 16
NW = NC * NS
E_PER_W = B // NW
CHUNK_E = 8
ROWS = CHUNK_E * L
N_CHUNKS = E_PER_W // CHUNK_E
IDXC = 4


def _emb_pool_sc(xw_flat, xb_flat, table, elem_lo, nb):
    e_per_w = nb // NW
    n_chunks = e_per_w // CHUNK_E
    mesh = plsc.VectorSubcoreMesh(core_axis_name="c", subcore_axis_name="s")

    @functools.partial(
        pl.kernel,
        out_type=jax.ShapeDtypeStruct((nb // CHUNK_E, 2, CHUNK_E, D),
                                      jnp.float32),
        mesh=mesh,
        scratch_types=[
            pltpu.VMEM((IDXC * ROWS,), jnp.int32),
            pltpu.VMEM((IDXC * ROWS,), jnp.int32),
            pltpu.VMEM((ROWS, D), jnp.float32),
            pltpu.VMEM((ROWS, D), jnp.float32),
            pltpu.VMEM((2, CHUNK_E, D), jnp.float32),
            pltpu.SemaphoreType.DMA,
            pltpu.SemaphoreType.DMA,
        ],
    )
    def k(xw_hbm, xb_hbm, table_hbm, out_hbm,
          idx0, idx1, rows0, rows1, outc_v, sem0, sem1):
        wid = lax.axis_index("s") * NC + lax.axis_index("c")
        w_base = elem_lo + wid * e_per_w
        idx_b, rows_b, sems = [idx0, idx1], [rows0, rows1], [sem0, sem1]
        srcs = [xw_hbm, xb_hbm]

        def fire(buf, chunk):
            @pl.when(chunk % IDXC == 0)
            def _():
                idx_base = (w_base + chunk * CHUNK_E) * L
                pltpu.sync_copy(
                    srcs[buf].at[pl.ds(idx_base, IDXC * ROWS)], idx_b[buf])

            slot = (chunk % IDXC) * ROWS
            off = 0
            while off < ROWS:
                n = min(128, ROWS - off)
                pltpu.async_copy(
                    table_hbm.at[idx_b[buf].at[pl.ds(slot + off, n)]],
                    rows_b[buf].at[pl.ds(off, n)], sems[buf])
                off += n

        def drain(buf):
            pltpu.make_async_copy(
                table_hbm.at[pl.ds(0, ROWS)], rows_b[buf], sems[buf]).wait()

        def unpack_row(rows_v, r):
            return tuple(rows_v[r, pl.ds(d * 16, 16)]
                         for d in range(D // 16))

        def reduce_side(buf):
            rows_v = rows_b[buf]
            for e in range(CHUNK_E):
                r0 = e * L

                def body7(t, acc, r0=r0, rows_v=rows_v):
                    j = 1 + t * 7
                    for u in range(7):
                        vals = unpack_row(rows_v, r0 + j + u)
                        acc = tuple(a + v for a, v in zip(acc, vals))
                    return acc

                acc = unpack_row(rows_v, r0)
                acc = lax.fori_loop(0, (L - 1) // 7, body7, acc)
                for d in range(D // 16):
                    outc_v[buf, e, pl.ds(d * 16, 16)] = acc[d]

        fire(0, 0)

        def chunk_body(c, carry):
            fire(1, c)
            drain(0)
            reduce_side(0)
            fire(0, jnp.minimum(c + 1, n_chunks - 1))
            drain(1)
            reduce_side(1)
            pltpu.sync_copy(outc_v, out_hbm.at[wid * n_chunks + c])
            return carry

        lax.fori_loop(0, n_chunks, chunk_body, 0)
        drain(0)

    return k(xw_flat, xb_flat, table)


def _mlp_tc(x4, W2, b2, W3, b3, W4, b4):
    BLK = 2048
    BC = BLK // CHUNK_E

    def body(x_ref, w2_ref, b2_ref, w3_ref, b3_ref, w4_ref, b4_ref, o_ref):
        xw = jnp.maximum(x_ref[:, 0].reshape(BLK, D), 0.0)
        xb = jnp.maximum(x_ref[:, 1].reshape(BLK, D), 0.0)
        h = (jnp.dot(xw, w2_ref[:D], preferred_element_type=jnp.float32)
             + jnp.dot(xb, w2_ref[D:], preferred_element_type=jnp.float32))
        h = jnp.maximum(h + b2_ref[...], 0.0)
        h = jnp.dot(h, w3_ref[...], preferred_element_type=jnp.float32)
        h = jnp.maximum(h + b3_ref[...], 0.0)
        h = jnp.dot(h, w4_ref[...], preferred_element_type=jnp.float32)
        o_ref[...] = h + b4_ref[...]

    nb = x4.shape[0] * CHUNK_E
    return pl.pallas_call(
        body,
        grid=(nb // BLK,),
        in_specs=[
            pl.BlockSpec((BC, 2, CHUNK_E, D), lambda i: (i, 0, 0, 0)),
            pl.BlockSpec((2 * D, 32), lambda i: (0, 0)),
            pl.BlockSpec((1, 32), lambda i: (0, 0)),
            pl.BlockSpec((32, 32), lambda i: (0, 0)),
            pl.BlockSpec((1, 32), lambda i: (0, 0)),
            pl.BlockSpec((32, 1), lambda i: (0, 0)),
            pl.BlockSpec((1, 1), lambda i: (0, 0)),
        ],
        out_specs=pl.BlockSpec((BLK, 1), lambda i: (i, 0)),
        out_shape=jax.ShapeDtypeStruct((nb, 1), jnp.float32),
    )(x4, W2, b2.reshape(1, 32), W3, b3.reshape(1, 32), W4, b4.reshape(1, 1))


def kernel(x_w, x_b, table, W2, b2, W3, b3, W4, b4):
    xw_flat = x_w.astype(jnp.int32).reshape(-1)
    xb_flat = x_b.astype(jnp.int32).reshape(-1)
    halves = []
    for h in range(2):
        pooled = _emb_pool_sc(xw_flat, xb_flat, table,
                              h * (B // 2), B // 2)
        halves.append(_mlp_tc(pooled, W2, b2, W3, b3, W4, b4))
    return jnp.concatenate(halves, axis=0)
